# baseline (speedup 1.0000x reference)
import functools

import jax
import jax.numpy as jnp
from jax import lax
from jax.experimental import pallas as pl
from jax.experimental.pallas import tpu as pltpu
from jax.experimental.pallas import tpu_sc as plsc

N = 10000
E = 320000
H = 128
DOUT = 3

NC = 2
NS = 16
NW = NC * NS
EPW = E // NW
CH = 200
CHS = 80
NPAD = 10240
RPT = NPAD // NS

BN = 2000
BE = 4000
E2 = E // 2
BE2 = 2000

_f32 = jnp.float32


def _row_spec(blk, w):
    return pl.BlockSpec((blk, w), lambda i: (i, 0))


def _rep_spec(r, c):
    return pl.BlockSpec((r, c), lambda i: (0, 0))


def _ln(t, g, b):
    m = jnp.mean(t, axis=-1, keepdims=True)
    v = jnp.mean((t - m) * (t - m), axis=-1, keepdims=True)
    return (t - m) * lax.rsqrt(v + 1e-5) * g + b


def _dot(a, w):
    return jnp.dot(a, w, preferred_element_type=_f32)


def _bdot(a, w):
    return jnp.dot(a.astype(jnp.bfloat16), w.astype(jnp.bfloat16),
                   preferred_element_type=_f32)


def _node_enc_body(x, mean, std, w0, b0, w1, b1, g, b, o):
    xn = (x[...] - mean[...]) / std[...]
    h = jnp.maximum(_dot(xn, w0[...]) + b0[...], 0.0)
    o[...] = _ln(_dot(h, w1[...]) + b1[...], g[...], b[...])


def _edge_enc_body(ea, mean, std, w0, b0, w1, b1, g, b, o):
    ean = (ea[...] - mean[...]) / std[...]
    h = jnp.maximum(_bdot(ean, w0[...]) + b0[...], 0.0)
    o[...] = _ln(_bdot(h, w1[...]) + b1[...], g[...], b[...])


def _proj_body(x, wd, ws, oa, ob):
    oa[...] = _dot(x[...], wd[...])
    ob[...] = _dot(x[...], ws[...])


def _edge_upd_body(g1, ea, we, b0, w1, b1, g, b, o):
    pre = g1[...] + _bdot(ea[...], we[...]) + b0[...]
    h = jnp.maximum(pre, 0.0)
    o[...] = _ln(_bdot(h, w1[...]) + b1[...], g[...], b[...]) + ea[...]


def _node_upd_body(x, p0, p1, wx, wa, b0, w1, b1, g, b, o):
    agg = p0[...] + p1[...]
    h = jnp.maximum(_dot(x[...], wx[...]) + _dot(agg, wa[...]) + b0[...], 0.0)
    o[...] = x[...] + _ln(_dot(h, w1[...]) + b1[...], g[...], b[...])


def _dec_body(x, w0, b0, w1, b1, o):
    h = jnp.maximum(_dot(x[...], w0[...]) + b0[...], 0.0)
    o[...] = _dot(h, w1[...]) + b1[...]


def _mlp_w(p):
    return (p["l0"]["w"], p["l0"]["b"].reshape(1, -1),
            p["l1"]["w"], p["l1"]["b"].reshape(1, -1))


def _node_enc(x, mean, std, p):
    w0, b0, w1, b1 = _mlp_w(p)
    g, b = p["ln"]["g"].reshape(1, H), p["ln"]["b"].reshape(1, H)
    return pl.pallas_call(
        _node_enc_body,
        grid=(N // BN,),
        in_specs=[_row_spec(BN, H), _rep_spec(1, H), _rep_spec(1, H),
                  _rep_spec(H, H), _rep_spec(1, H), _rep_spec(H, H),
                  _rep_spec(1, H), _rep_spec(1, H), _rep_spec(1, H)],
        out_specs=_row_spec(BN, H),
        out_shape=jax.ShapeDtypeStruct((N, H), _f32),
    )(x, mean.reshape(1, H), std.reshape(1, H), w0, b0, w1, b1, g, b)


def _edge_enc(ea, mean, std, p):
    w0, b0, w1, b1 = _mlp_w(p)
    g, b = p["ln"]["g"].reshape(1, H), p["ln"]["b"].reshape(1, H)
    din = ea.shape[1]
    return pl.pallas_call(
        _edge_enc_body,
        grid=(E // BE,),
        in_specs=[_row_spec(BE, din), _rep_spec(1, din), _rep_spec(1, din),
                  _rep_spec(din, H), _rep_spec(1, H), _rep_spec(H, H),
                  _rep_spec(1, H), _rep_spec(1, H), _rep_spec(1, H)],
        out_specs=_row_spec(BE, H),
        out_shape=jax.ShapeDtypeStruct((E, H), _f32),
    )(ea, mean.reshape(1, din), std.reshape(1, din), w0, b0, w1, b1, g, b)


def _proj(x, wd, ws):
    return pl.pallas_call(
        _proj_body,
        grid=(N // BN,),
        in_specs=[_row_spec(BN, H), _rep_spec(H, H), _rep_spec(H, H)],
        out_specs=(_row_spec(BN, H), _row_spec(BN, H)),
        out_shape=(jax.ShapeDtypeStruct((N, H), _f32),
                   jax.ShapeDtypeStruct((N, H), _f32)),
    )(x, wd, ws)


def _edge_upd(g1, ea, we, b0, w1, b1, g, b):
    return pl.pallas_call(
        _edge_upd_body,
        grid=(E // BE,),
        in_specs=[_row_spec(BE, H), _row_spec(BE, H),
                  _rep_spec(H, H), _rep_spec(1, H), _rep_spec(H, H),
                  _rep_spec(1, H), _rep_spec(1, H), _rep_spec(1, H)],
        out_specs=_row_spec(BE, H),
        out_shape=jax.ShapeDtypeStruct((E, H), _f32),
    )(g1, ea, we, b0, w1, b1, g, b)


def _node_upd(x, p0, p1, wx, wa, b0, w1, b1, g, b):
    return pl.pallas_call(
        _node_upd_body,
        grid=(N // BN,),
        in_specs=[_row_spec(BN, H)] * 3 +
                 [_rep_spec(H, H), _rep_spec(H, H), _rep_spec(1, H),
                  _rep_spec(H, H), _rep_spec(1, H), _rep_spec(1, H),
                  _rep_spec(1, H)],
        out_specs=_row_spec(BN, H),
        out_shape=jax.ShapeDtypeStruct((N, H), _f32),
    )(x, p0, p1, wx, wa, b0, w1, b1, g, b)


def _dec(x, p):
    w0, b0, w1, b1 = _mlp_w(p)
    return pl.pallas_call(
        _dec_body,
        grid=(N // BN,),
        in_specs=[_row_spec(BN, H), _rep_spec(H, H), _rep_spec(1, H),
                  _rep_spec(H, DOUT), _rep_spec(1, DOUT)],
        out_specs=_row_spec(BN, DOUT),
        out_shape=jax.ShapeDtypeStruct((N, DOUT), _f32),
    )(x, w0, b0, w1, b1)


def _sc_gather(a, b, dst, src):
    mesh = plsc.VectorSubcoreMesh(core_axis_name="c", subcore_axis_name="s")
    nit = EPW // CH
    npair = nit // 2

    @functools.partial(
        pl.kernel,
        out_type=jax.ShapeDtypeStruct((E, H), _f32),
        mesh=mesh,
        scratch_types=[
            pltpu.VMEM((CH,), jnp.int32),
            pltpu.VMEM((CH,), jnp.int32),
            pltpu.VMEM((CH,), jnp.int32),
            pltpu.VMEM((CH,), jnp.int32),
            pltpu.VMEM((CH, H), _f32),
            pltpu.VMEM((CH, H), _f32),
            pltpu.VMEM((CH, H), _f32),
            pltpu.VMEM((CH, H), _f32),
            pltpu.SemaphoreType.DMA((2,)),
            pltpu.SemaphoreType.DMA((2,)),
            pltpu.SemaphoreType.DMA((2,)),
            pltpu.SemaphoreType.DMA((2,)),
            pltpu.SemaphoreType.DMA((2,)),
        ],
    )
    def k(a_hbm, b_hbm, dst_hbm, src_hbm, g_hbm,
          ia0, ia1, ib0, ib1, ba0, ba1, bb0, bb1,
          sem_ia, sem_ib, sem_ga, sem_gb, sem_o):
        wid = lax.axis_index("s") * NC + lax.axis_index("c")
        base = wid * EPW
        idx_a = (ia0, ia1)
        idx_b = (ib0, ib1)
        buf_a = (ba0, ba1)
        buf_b = (bb0, bb1)

        def idx_cp(g, j):
            off = base + g * CH
            return (pltpu.make_async_copy(dst_hbm.at[pl.ds(off, CH)],
                                          idx_a[j], sem_ia.at[j]),
                    pltpu.make_async_copy(src_hbm.at[pl.ds(off, CH)],
                                          idx_b[j], sem_ib.at[j]))

        def gat_cp(j):
            return (pltpu.make_async_copy(a_hbm.at[idx_a[j]],
                                          buf_a[j], sem_ga.at[j]),
                    pltpu.make_async_copy(b_hbm.at[idx_b[j]],
                                          buf_b[j], sem_gb.at[j]))

        def out_cp(g, j):
            off = base + g * CH
            return (pltpu.make_async_copy(buf_a[j],
                                          g_hbm.at[pl.ds(off, CH)],
                                          sem_o.at[j]),)

        def start(cps):
            for cp in cps:
                cp.start()

        def wait(cps):
            for cp in cps:
                cp.wait()

        def add_inplace(j):
            av = buf_a[j]
            bv = buf_b[j]

            def row(r, carry):
                for cseg in range(H // 16):
                    sl = pl.ds(cseg * 16, 16)
                    av[r, sl] = av[r, sl] + bv[r, sl]
                return carry

            lax.fori_loop(0, CH, row, 0)

        start(idx_cp(0, 0))
        start(idx_cp(1, 1))
        wait(idx_cp(0, 0))
        start(gat_cp(0))
        wait(idx_cp(1, 1))
        start(gat_cp(1))

        def body(i, carry):
            g0 = 2 * i
            g1 = g0 + 1

            wait(gat_cp(0))
            @pl.when(g0 + 2 < nit)
            def _():
                start(idx_cp(g0 + 2, 0))
            add_inplace(0)
            start(out_cp(g0, 0))

            wait(gat_cp(1))
            @pl.when(g1 + 2 < nit)
            def _():
                start(idx_cp(g1 + 2, 1))
            add_inplace(1)
            start(out_cp(g1, 1))

            wait(out_cp(g0, 0))
            @pl.when(g0 + 2 < nit)
            def _():
                wait(idx_cp(g0 + 2, 0))
                start(gat_cp(0))
            wait(out_cp(g1, 1))
            @pl.when(g1 + 2 < nit)
            def _():
                wait(idx_cp(g1 + 2, 1))
                start(gat_cp(1))
            return carry

        lax.fori_loop(0, npair, body, 0)

    return k(a, b, dst, src)


def _sc_scatter(ue, src, zrows):
    mesh = plsc.VectorSubcoreMesh(core_axis_name="c", subcore_axis_name="s")
    nit = EPW // CHS
    npair = nit // 2

    @functools.partial(
        pl.kernel,
        out_type=jax.ShapeDtypeStruct((NC, NPAD, H), _f32),
        mesh=mesh,
        scratch_types=[
            pltpu.VMEM((CHS,), jnp.int32),
            pltpu.VMEM((CHS,), jnp.int32),
            pltpu.VMEM((CHS, H), _f32),
            pltpu.VMEM((CHS, H), _f32),
            pltpu.VMEM_SHARED((NPAD, H), _f32),
            pltpu.SemaphoreType.DMA((2,)),
            pltpu.SemaphoreType.DMA((2,)),
        ],
    )
    def k(ue_hbm, src_hbm, z_hbm, out_hbm, ix0, ix1, bf0, bf1, acc_sh,
          sem_i, sem_u):
        c = lax.axis_index("c")
        s = lax.axis_index("s")
        r0 = s * RPT
        pltpu.sync_copy(z_hbm, acc_sh.at[pl.ds(r0, RPT)])
        plsc.subcore_barrier()
        base = c * (E // NC) + s * EPW
        idx_v = (ix0, ix1)
        buf = (bf0, bf1)

        def load_cp(g, j):
            return (pltpu.make_async_copy(src_hbm.at[pl.ds(base + g * CHS, CHS)],
                                          idx_v[j], sem_i.at[j]),
                    pltpu.make_async_copy(ue_hbm.at[pl.ds(base + g * CHS, CHS)],
                                          buf[j], sem_u.at[j]))

        def start(cps):
            for cp in cps:
                cp.start()

        def wait(cps):
            for cp in cps:
                cp.wait()

        start(load_cp(0, 0))

        def body(i, carry):
            g0 = 2 * i
            g1 = g0 + 1
            start(load_cp(g1, 1))
            wait(load_cp(g0, 0))
            pltpu.sync_copy(buf[0], acc_sh.at[idx_v[0]], add=True)

            @pl.when(g1 + 1 < nit)
            def _():
                start(load_cp(g1 + 1, 0))
            wait(load_cp(g1, 1))
            pltpu.sync_copy(buf[1], acc_sh.at[idx_v[1]], add=True)
            return carry

        lax.fori_loop(0, npair, body, 0)
        if nit % 2 == 1:
            wait(load_cp(nit - 1, 0))
            pltpu.sync_copy(buf[0], acc_sh.at[idx_v[0]], add=True)
        plsc.subcore_barrier()
        pltpu.sync_copy(acc_sh.at[pl.ds(r0, RPT)],
                        out_hbm.at[c, pl.ds(r0, RPT)])

    return k(ue, src, zrows)


def kernel(x, edge_index, edge_attr, mean_vec_x, std_vec_x,
           mean_vec_edge, std_vec_edge, params):
    src = edge_index[0].astype(jnp.int32)
    dst = edge_index[1].astype(jnp.int32)

    xc = _node_enc(x, mean_vec_x, std_vec_x, params["node_enc"])
    ea = _edge_enc(edge_attr, mean_vec_edge, std_vec_edge, params["edge_enc"])
    zrows = jnp.zeros((RPT, H), _f32)

    for lp in params["layers"]:
        ew = lp["edge_mlp"]
        w0e = ew["l0"]["w"]
        wd, ws, we = w0e[:H], w0e[H:2 * H], w0e[2 * H:]
        b0e = ew["l0"]["b"].reshape(1, H)
        w1e, b1e = ew["l1"]["w"], ew["l1"]["b"].reshape(1, H)
        ge, be = ew["ln"]["g"].reshape(1, H), ew["ln"]["b"].reshape(1, H)

        a, b = _proj(xc, wd, ws)
        gsum = _sc_gather(a, b, dst, src)
        ue = _edge_upd(gsum, ea, we, b0e, w1e, b1e, ge, be)
        parts = _sc_scatter(ue, src, zrows)
        p0, p1 = parts[0, :N], parts[1, :N]

        nw = lp["node_mlp"]
        w0n = nw["l0"]["w"]
        wx, wa = w0n[:H], w0n[H:]
        b0n = nw["l0"]["b"].reshape(1, H)
        w1n, b1n = nw["l1"]["w"], nw["l1"]["b"].reshape(1, H)
        gn, bn = nw["ln"]["g"].reshape(1, H), nw["ln"]["b"].reshape(1, H)
        xc = _node_upd(xc, p0, p1, wx, wa, b0n, w1n, b1n, gn, bn)
        ea = ue

    return _dec(xc, params["dec"])


# speedup vs baseline: 1.0527x; 1.0527x over previous
import functools

import jax
import jax.numpy as jnp
from jax import lax
from jax.experimental import pallas as pl
from jax.experimental.pallas import tpu as pltpu
from jax.experimental.pallas import tpu_sc as plsc

N = 10000
E = 320000
H = 128
DOUT = 3

NC = 2
NS = 16
NW = NC * NS
EPW = E // NW
CH = 200
CHS = 80
NPAD = 10240
RPT = NPAD // NS

BN = 2000
BE = 4000
E2 = E // 2
BE2 = 2000

_f32 = jnp.float32


def _row_spec(blk, w):
    return pl.BlockSpec((blk, w), lambda i: (i, 0))


def _rep_spec(r, c):
    return pl.BlockSpec((r, c), lambda i: (0, 0))


def _ln(t, g, b):
    m = jnp.mean(t, axis=-1, keepdims=True)
    v = jnp.mean((t - m) * (t - m), axis=-1, keepdims=True)
    return (t - m) * lax.rsqrt(v + 1e-5) * g + b


def _dot(a, w):
    return jnp.dot(a, w, preferred_element_type=_f32)


def _bdot(a, w):
    return jnp.dot(a.astype(jnp.bfloat16), w.astype(jnp.bfloat16),
                   preferred_element_type=_f32)


def _node_enc_body(x, mean, std, w0, b0, w1, b1, g, b, o):
    xn = (x[...] - mean[...]) / std[...]
    h = jnp.maximum(_dot(xn, w0[...]) + b0[...], 0.0)
    o[...] = _ln(_dot(h, w1[...]) + b1[...], g[...], b[...])


def _edge_enc_body(ea, mean, std, w0, b0, w1, b1, g, b, o):
    ean = (ea[...] - mean[...]) / std[...]
    h = jnp.maximum(_dot(ean, w0[...]) + b0[...], 0.0)
    o[...] = _ln(_dot(h, w1[...]) + b1[...], g[...], b[...])


def _proj_body(x, wd, ws, oa, ob):
    oa[...] = _dot(x[...], wd[...])
    ob[...] = _dot(x[...], ws[...])


def _edge_upd_body(g1, ea, we, b0, w1, b1, g, b, o):
    pre = g1[...] + _dot(ea[...], we[...]) + b0[...]
    h = jnp.maximum(pre, 0.0)
    o[...] = _ln(_dot(h, w1[...]) + b1[...], g[...], b[...]) + ea[...]


def _edge_upd0_body(ea4, mean, std, ew0, eb0, ew1, eb1, eg, eb,
                    g1, we, b0, w1, b1, g, b, o):
    ean = (ea4[...] - mean[...]) / std[...]
    eh = jnp.maximum(_dot(ean, ew0[...]) + eb0[...], 0.0)
    ea = _ln(_dot(eh, ew1[...]) + eb1[...], eg[...], eb[...])
    pre = g1[...] + _dot(ea, we[...]) + b0[...]
    h = jnp.maximum(pre, 0.0)
    o[...] = _ln(_dot(h, w1[...]) + b1[...], g[...], b[...]) + ea


def _node_upd_body(x, p0, p1, wx, wa, b0, w1, b1, g, b, o):
    agg = p0[...] + p1[...]
    h = jnp.maximum(_dot(x[...], wx[...]) + _dot(agg, wa[...]) + b0[...], 0.0)
    o[...] = x[...] + _ln(_dot(h, w1[...]) + b1[...], g[...], b[...])


def _dec_body(x, w0, b0, w1, b1, o):
    h = jnp.maximum(_dot(x[...], w0[...]) + b0[...], 0.0)
    o[...] = _dot(h, w1[...]) + b1[...]


def _mlp_w(p):
    return (p["l0"]["w"], p["l0"]["b"].reshape(1, -1),
            p["l1"]["w"], p["l1"]["b"].reshape(1, -1))


def _node_enc(x, mean, std, p):
    w0, b0, w1, b1 = _mlp_w(p)
    g, b = p["ln"]["g"].reshape(1, H), p["ln"]["b"].reshape(1, H)
    return pl.pallas_call(
        _node_enc_body,
        grid=(N // BN,),
        in_specs=[_row_spec(BN, H), _rep_spec(1, H), _rep_spec(1, H),
                  _rep_spec(H, H), _rep_spec(1, H), _rep_spec(H, H),
                  _rep_spec(1, H), _rep_spec(1, H), _rep_spec(1, H)],
        out_specs=_row_spec(BN, H),
        out_shape=jax.ShapeDtypeStruct((N, H), _f32),
    )(x, mean.reshape(1, H), std.reshape(1, H), w0, b0, w1, b1, g, b)


def _edge_enc(ea, mean, std, p):
    w0, b0, w1, b1 = _mlp_w(p)
    g, b = p["ln"]["g"].reshape(1, H), p["ln"]["b"].reshape(1, H)
    din = ea.shape[1]
    return pl.pallas_call(
        _edge_enc_body,
        grid=(E // BE,),
        in_specs=[_row_spec(BE, din), _rep_spec(1, din), _rep_spec(1, din),
                  _rep_spec(din, H), _rep_spec(1, H), _rep_spec(H, H),
                  _rep_spec(1, H), _rep_spec(1, H), _rep_spec(1, H)],
        out_specs=_row_spec(BE, H),
        out_shape=jax.ShapeDtypeStruct((E, H), _f32),
    )(ea, mean.reshape(1, din), std.reshape(1, din), w0, b0, w1, b1, g, b)


def _proj(x, wd, ws):
    return pl.pallas_call(
        _proj_body,
        grid=(N // BN,),
        in_specs=[_row_spec(BN, H), _rep_spec(H, H), _rep_spec(H, H)],
        out_specs=(_row_spec(BN, H), _row_spec(BN, H)),
        out_shape=(jax.ShapeDtypeStruct((N, H), _f32),
                   jax.ShapeDtypeStruct((N, H), _f32)),
    )(x, wd, ws)


def _edge_upd(g1, ea, we, b0, w1, b1, g, b):
    return pl.pallas_call(
        _edge_upd_body,
        grid=(E // BE,),
        in_specs=[_row_spec(BE, H), _row_spec(BE, H),
                  _rep_spec(H, H), _rep_spec(1, H), _rep_spec(H, H),
                  _rep_spec(1, H), _rep_spec(1, H), _rep_spec(1, H)],
        out_specs=_row_spec(BE, H),
        out_shape=jax.ShapeDtypeStruct((E, H), _f32),
    )(g1, ea, we, b0, w1, b1, g, b)


def _edge_upd0(ea4, mean, std, encp, we, b0, w1, b1, g, b):
    ew0, eb0, ew1, eb1 = _mlp_w(encp)
    eg = encp["ln"]["g"].reshape(1, H)
    eb = encp["ln"]["b"].reshape(1, H)
    din = ea4.shape[1]

    def call(g1):
        return pl.pallas_call(
            _edge_upd0_body,
            grid=(E // BE,),
            in_specs=[_row_spec(BE, din), _rep_spec(1, din),
                      _rep_spec(1, din), _rep_spec(din, H), _rep_spec(1, H),
                      _rep_spec(H, H), _rep_spec(1, H), _rep_spec(1, H),
                      _rep_spec(1, H),
                      _row_spec(BE, H),
                      _rep_spec(H, H), _rep_spec(1, H), _rep_spec(H, H),
                      _rep_spec(1, H), _rep_spec(1, H), _rep_spec(1, H)],
            out_specs=_row_spec(BE, H),
            out_shape=jax.ShapeDtypeStruct((E, H), _f32),
        )(ea4, mean.reshape(1, din), std.reshape(1, din),
          ew0, eb0, ew1, eb1, eg, eb,
          g1, we, b0, w1, b1, g, b)

    return call


def _node_upd(x, p0, p1, wx, wa, b0, w1, b1, g, b):
    return pl.pallas_call(
        _node_upd_body,
        grid=(N // BN,),
        in_specs=[_row_spec(BN, H)] * 3 +
                 [_rep_spec(H, H), _rep_spec(H, H), _rep_spec(1, H),
                  _rep_spec(H, H), _rep_spec(1, H), _rep_spec(1, H),
                  _rep_spec(1, H)],
        out_specs=_row_spec(BN, H),
        out_shape=jax.ShapeDtypeStruct((N, H), _f32),
    )(x, p0, p1, wx, wa, b0, w1, b1, g, b)


def _dec(x, p):
    w0, b0, w1, b1 = _mlp_w(p)
    return pl.pallas_call(
        _dec_body,
        grid=(N // BN,),
        in_specs=[_row_spec(BN, H), _rep_spec(H, H), _rep_spec(1, H),
                  _rep_spec(H, DOUT), _rep_spec(1, DOUT)],
        out_specs=_row_spec(BN, DOUT),
        out_shape=jax.ShapeDtypeStruct((N, DOUT), _f32),
    )(x, w0, b0, w1, b1)


def _sc_gather(a, b, dst, src):
    mesh = plsc.VectorSubcoreMesh(core_axis_name="c", subcore_axis_name="s")
    nit = EPW // CH
    npair = nit // 2

    @functools.partial(
        pl.kernel,
        out_type=jax.ShapeDtypeStruct((E, H), _f32),
        mesh=mesh,
        scratch_types=[
            pltpu.VMEM((CH,), jnp.int32),
            pltpu.VMEM((CH,), jnp.int32),
            pltpu.VMEM((CH,), jnp.int32),
            pltpu.VMEM((CH,), jnp.int32),
            pltpu.VMEM((CH, H), _f32),
            pltpu.VMEM((CH, H), _f32),
            pltpu.VMEM((CH, H), _f32),
            pltpu.VMEM((CH, H), _f32),
            pltpu.SemaphoreType.DMA((2,)),
            pltpu.SemaphoreType.DMA((2,)),
            pltpu.SemaphoreType.DMA((2,)),
            pltpu.SemaphoreType.DMA((2,)),
            pltpu.SemaphoreType.DMA((2,)),
        ],
    )
    def k(a_hbm, b_hbm, dst_hbm, src_hbm, g_hbm,
          ia0, ia1, ib0, ib1, ba0, ba1, bb0, bb1,
          sem_ia, sem_ib, sem_ga, sem_gb, sem_o):
        wid = lax.axis_index("s") * NC + lax.axis_index("c")
        base = wid * EPW
        idx_a = (ia0, ia1)
        idx_b = (ib0, ib1)
        buf_a = (ba0, ba1)
        buf_b = (bb0, bb1)

        def idx_cp(g, j):
            off = base + g * CH
            return (pltpu.make_async_copy(dst_hbm.at[pl.ds(off, CH)],
                                          idx_a[j], sem_ia.at[j]),
                    pltpu.make_async_copy(src_hbm.at[pl.ds(off, CH)],
                                          idx_b[j], sem_ib.at[j]))

        def gat_cp(j):
            return (pltpu.make_async_copy(a_hbm.at[idx_a[j]],
                                          buf_a[j], sem_ga.at[j]),
                    pltpu.make_async_copy(b_hbm.at[idx_b[j]],
                                          buf_b[j], sem_gb.at[j]))

        def out_cp(g, j):
            off = base + g * CH
            return (pltpu.make_async_copy(buf_a[j],
                                          g_hbm.at[pl.ds(off, CH)],
                                          sem_o.at[j]),)

        def start(cps):
            for cp in cps:
                cp.start()

        def wait(cps):
            for cp in cps:
                cp.wait()

        def add_inplace(j):
            av = buf_a[j]
            bv = buf_b[j]

            def row(r, carry):
                for cseg in range(H // 16):
                    sl = pl.ds(cseg * 16, 16)
                    av[r, sl] = av[r, sl] + bv[r, sl]
                return carry

            lax.fori_loop(0, CH, row, 0)

        start(idx_cp(0, 0))
        start(idx_cp(1, 1))
        wait(idx_cp(0, 0))
        start(gat_cp(0))
        wait(idx_cp(1, 1))
        start(gat_cp(1))

        def body(i, carry):
            g0 = 2 * i
            g1 = g0 + 1

            wait(gat_cp(0))
            @pl.when(g0 + 2 < nit)
            def _():
                start(idx_cp(g0 + 2, 0))
            add_inplace(0)
            start(out_cp(g0, 0))

            wait(gat_cp(1))
            @pl.when(g1 + 2 < nit)
            def _():
                start(idx_cp(g1 + 2, 1))
            add_inplace(1)
            start(out_cp(g1, 1))

            wait(out_cp(g0, 0))
            @pl.when(g0 + 2 < nit)
            def _():
                wait(idx_cp(g0 + 2, 0))
                start(gat_cp(0))
            wait(out_cp(g1, 1))
            @pl.when(g1 + 2 < nit)
            def _():
                wait(idx_cp(g1 + 2, 1))
                start(gat_cp(1))
            return carry

        lax.fori_loop(0, npair, body, 0)

    return k(a, b, dst, src)


def _sc_scatter(ue, src, zrows):
    mesh = plsc.VectorSubcoreMesh(core_axis_name="c", subcore_axis_name="s")
    nit = EPW // CHS
    npair = nit // 2

    @functools.partial(
        pl.kernel,
        out_type=jax.ShapeDtypeStruct((NC, NPAD, H), _f32),
        mesh=mesh,
        scratch_types=[
            pltpu.VMEM((CHS,), jnp.int32),
            pltpu.VMEM((CHS,), jnp.int32),
            pltpu.VMEM((CHS, H), _f32),
            pltpu.VMEM((CHS, H), _f32),
            pltpu.VMEM_SHARED((NPAD, H), _f32),
            pltpu.SemaphoreType.DMA((2,)),
            pltpu.SemaphoreType.DMA((2,)),
        ],
    )
    def k(ue_hbm, src_hbm, z_hbm, out_hbm, ix0, ix1, bf0, bf1, acc_sh,
          sem_i, sem_u):
        c = lax.axis_index("c")
        s = lax.axis_index("s")
        r0 = s * RPT
        pltpu.sync_copy(z_hbm, acc_sh.at[pl.ds(r0, RPT)])
        plsc.subcore_barrier()
        base = c * (E // NC) + s * EPW
        idx_v = (ix0, ix1)
        buf = (bf0, bf1)

        def load_cp(g, j):
            return (pltpu.make_async_copy(src_hbm.at[pl.ds(base + g * CHS, CHS)],
                                          idx_v[j], sem_i.at[j]),
                    pltpu.make_async_copy(ue_hbm.at[pl.ds(base + g * CHS, CHS)],
                                          buf[j], sem_u.at[j]))

        def start(cps):
            for cp in cps:
                cp.start()

        def wait(cps):
            for cp in cps:
                cp.wait()

        start(load_cp(0, 0))

        def body(i, carry):
            g0 = 2 * i
            g1 = g0 + 1
            start(load_cp(g1, 1))
            wait(load_cp(g0, 0))
            pltpu.sync_copy(buf[0], acc_sh.at[idx_v[0]], add=True)

            @pl.when(g1 + 1 < nit)
            def _():
                start(load_cp(g1 + 1, 0))
            wait(load_cp(g1, 1))
            pltpu.sync_copy(buf[1], acc_sh.at[idx_v[1]], add=True)
            return carry

        lax.fori_loop(0, npair, body, 0)
        if nit % 2 == 1:
            wait(load_cp(nit - 1, 0))
            pltpu.sync_copy(buf[0], acc_sh.at[idx_v[0]], add=True)
        plsc.subcore_barrier()
        pltpu.sync_copy(acc_sh.at[pl.ds(r0, RPT)],
                        out_hbm.at[c, pl.ds(r0, RPT)])

    return k(ue, src, zrows)


def kernel(x, edge_index, edge_attr, mean_vec_x, std_vec_x,
           mean_vec_edge, std_vec_edge, params):
    src = edge_index[0].astype(jnp.int32)
    dst = edge_index[1].astype(jnp.int32)

    xc = _node_enc(x, mean_vec_x, std_vec_x, params["node_enc"])
    ea = None
    zrows = jnp.zeros((RPT, H), _f32)

    for li, lp in enumerate(params["layers"]):
        ew = lp["edge_mlp"]
        w0e = ew["l0"]["w"]
        wd, ws, we = w0e[:H], w0e[H:2 * H], w0e[2 * H:]
        b0e = ew["l0"]["b"].reshape(1, H)
        w1e, b1e = ew["l1"]["w"], ew["l1"]["b"].reshape(1, H)
        ge, be = ew["ln"]["g"].reshape(1, H), ew["ln"]["b"].reshape(1, H)

        a, b = _proj(xc, wd, ws)
        gsum = _sc_gather(a, b, dst, src)
        if li == 0:
            ue = _edge_upd0(edge_attr, mean_vec_edge, std_vec_edge,
                            params["edge_enc"], we, b0e, w1e, b1e,
                            ge, be)(gsum)
        else:
            ue = _edge_upd(gsum, ea, we, b0e, w1e, b1e, ge, be)
        parts = _sc_scatter(ue, src, zrows)
        p0, p1 = parts[0, :N], parts[1, :N]

        nw = lp["node_mlp"]
        w0n = nw["l0"]["w"]
        wx, wa = w0n[:H], w0n[H:]
        b0n = nw["l0"]["b"].reshape(1, H)
        w1n, b1n = nw["l1"]["w"], nw["l1"]["b"].reshape(1, H)
        gn, bn = nw["ln"]["g"].reshape(1, H), nw["ln"]["b"].reshape(1, H)
        xc = _node_upd(xc, p0, p1, wx, wa, b0n, w1n, b1n, gn, bn)
        ea = ue

    return _dec(xc, params["dec"])


# speedup vs baseline: 1.0653x; 1.0119x over previous
import functools

import jax
import jax.numpy as jnp
from jax import lax
from jax.experimental import pallas as pl
from jax.experimental.pallas import tpu as pltpu
from jax.experimental.pallas import tpu_sc as plsc

N = 10000
E = 320000
H = 128
DOUT = 3

NC = 2
NS = 16
NW = NC * NS
EPW = E // NW
CH = 200
CHS = 80
NPAD = 10240
RPT = NPAD // NS

BN = 2000
BE = 4000
E2 = E // 2
BE2 = 2000

_f32 = jnp.float32


def _row_spec(blk, w):
    return pl.BlockSpec((blk, w), lambda i: (i, 0))


def _rep_spec(r, c):
    return pl.BlockSpec((r, c), lambda i: (0, 0))


def _ln(t, g, b):
    m = jnp.mean(t, axis=-1, keepdims=True)
    v = jnp.mean((t - m) * (t - m), axis=-1, keepdims=True)
    return (t - m) * lax.rsqrt(v + 1e-5) * g + b


def _dot(a, w):
    return jnp.dot(a, w, preferred_element_type=_f32)


def _bdot(a, w):
    return jnp.dot(a.astype(jnp.bfloat16), w.astype(jnp.bfloat16),
                   preferred_element_type=_f32)


def _node_enc_body(x, mean, std, w0, b0, w1, b1, g, b, wd, ws,
                   o, oa, ob):
    xn = (x[...] - mean[...]) / std[...]
    h = jnp.maximum(_dot(xn, w0[...]) + b0[...], 0.0)
    xe = _ln(_dot(h, w1[...]) + b1[...], g[...], b[...])
    o[...] = xe
    oa[...] = _dot(xe, wd[...])
    ob[...] = _dot(xe, ws[...])


def _edge_enc_body(ea, mean, std, w0, b0, w1, b1, g, b, o):
    ean = (ea[...] - mean[...]) / std[...]
    h = jnp.maximum(_dot(ean, w0[...]) + b0[...], 0.0)
    o[...] = _ln(_dot(h, w1[...]) + b1[...], g[...], b[...])


def _edge_upd_body(g1, ea, we, b0, w1, b1, g, b, o):
    pre = g1[...] + _dot(ea[...], we[...]) + b0[...]
    h = jnp.maximum(pre, 0.0)
    o[...] = _ln(_dot(h, w1[...]) + b1[...], g[...], b[...]) + ea[...]


def _edge_upd0_body(ea4, mean, std, ew0, eb0, ew1, eb1, eg, eb,
                    g1, we, b0, w1, b1, g, b, o):
    ean = (ea4[...] - mean[...]) / std[...]
    eh = jnp.maximum(_dot(ean, ew0[...]) + eb0[...], 0.0)
    ea = _ln(_dot(eh, ew1[...]) + eb1[...], eg[...], eb[...])
    pre = g1[...] + _dot(ea, we[...]) + b0[...]
    h = jnp.maximum(pre, 0.0)
    o[...] = _ln(_dot(h, w1[...]) + b1[...], g[...], b[...]) + ea


def _node_upd_body(x, p0, p1, wx, wa, b0, w1, b1, g, b, wd, ws,
                   o, oa, ob):
    agg = p0[...] + p1[...]
    h = jnp.maximum(_dot(x[...], wx[...]) + _dot(agg, wa[...]) + b0[...], 0.0)
    xn = x[...] + _ln(_dot(h, w1[...]) + b1[...], g[...], b[...])
    o[...] = xn
    oa[...] = _dot(xn, wd[...])
    ob[...] = _dot(xn, ws[...])


def _dec_body(x, w0, b0, w1, b1, o):
    h = jnp.maximum(_dot(x[...], w0[...]) + b0[...], 0.0)
    o[...] = _dot(h, w1[...]) + b1[...]


def _mlp_w(p):
    return (p["l0"]["w"], p["l0"]["b"].reshape(1, -1),
            p["l1"]["w"], p["l1"]["b"].reshape(1, -1))


def _node_enc(x, mean, std, p, wd, ws):
    w0, b0, w1, b1 = _mlp_w(p)
    g, b = p["ln"]["g"].reshape(1, H), p["ln"]["b"].reshape(1, H)
    return pl.pallas_call(
        _node_enc_body,
        grid=(N // BN,),
        in_specs=[_row_spec(BN, H), _rep_spec(1, H), _rep_spec(1, H),
                  _rep_spec(H, H), _rep_spec(1, H), _rep_spec(H, H),
                  _rep_spec(1, H), _rep_spec(1, H), _rep_spec(1, H),
                  _rep_spec(H, H), _rep_spec(H, H)],
        out_specs=(_row_spec(BN, H), _row_spec(BN, H), _row_spec(BN, H)),
        out_shape=(jax.ShapeDtypeStruct((N, H), _f32),
                   jax.ShapeDtypeStruct((N, H), _f32),
                   jax.ShapeDtypeStruct((N, H), _f32)),
    )(x, mean.reshape(1, H), std.reshape(1, H), w0, b0, w1, b1, g, b,
      wd, ws)


def _edge_enc(ea, mean, std, p):
    w0, b0, w1, b1 = _mlp_w(p)
    g, b = p["ln"]["g"].reshape(1, H), p["ln"]["b"].reshape(1, H)
    din = ea.shape[1]
    return pl.pallas_call(
        _edge_enc_body,
        grid=(E // BE,),
        in_specs=[_row_spec(BE, din), _rep_spec(1, din), _rep_spec(1, din),
                  _rep_spec(din, H), _rep_spec(1, H), _rep_spec(H, H),
                  _rep_spec(1, H), _rep_spec(1, H), _rep_spec(1, H)],
        out_specs=_row_spec(BE, H),
        out_shape=jax.ShapeDtypeStruct((E, H), _f32),
    )(ea, mean.reshape(1, din), std.reshape(1, din), w0, b0, w1, b1, g, b)


def _edge_upd(g1, ea, we, b0, w1, b1, g, b):
    return pl.pallas_call(
        _edge_upd_body,
        grid=(E // BE,),
        in_specs=[_row_spec(BE, H), _row_spec(BE, H),
                  _rep_spec(H, H), _rep_spec(1, H), _rep_spec(H, H),
                  _rep_spec(1, H), _rep_spec(1, H), _rep_spec(1, H)],
        out_specs=_row_spec(BE, H),
        out_shape=jax.ShapeDtypeStruct((E, H), _f32),
    )(g1, ea, we, b0, w1, b1, g, b)


def _edge_upd0(ea4, mean, std, encp, we, b0, w1, b1, g, b):
    ew0, eb0, ew1, eb1 = _mlp_w(encp)
    eg = encp["ln"]["g"].reshape(1, H)
    eb = encp["ln"]["b"].reshape(1, H)
    din = ea4.shape[1]

    def call(g1):
        return pl.pallas_call(
            _edge_upd0_body,
            grid=(E // BE,),
            in_specs=[_row_spec(BE, din), _rep_spec(1, din),
                      _rep_spec(1, din), _rep_spec(din, H), _rep_spec(1, H),
                      _rep_spec(H, H), _rep_spec(1, H), _rep_spec(1, H),
                      _rep_spec(1, H),
                      _row_spec(BE, H),
                      _rep_spec(H, H), _rep_spec(1, H), _rep_spec(H, H),
                      _rep_spec(1, H), _rep_spec(1, H), _rep_spec(1, H)],
            out_specs=_row_spec(BE, H),
            out_shape=jax.ShapeDtypeStruct((E, H), _f32),
        )(ea4, mean.reshape(1, din), std.reshape(1, din),
          ew0, eb0, ew1, eb1, eg, eb,
          g1, we, b0, w1, b1, g, b)

    return call


def _node_upd(x, p0, p1, wx, wa, b0, w1, b1, g, b, wd, ws):
    return pl.pallas_call(
        _node_upd_body,
        grid=(N // BN,),
        in_specs=[_row_spec(BN, H)] * 3 +
                 [_rep_spec(H, H), _rep_spec(H, H), _rep_spec(1, H),
                  _rep_spec(H, H), _rep_spec(1, H), _rep_spec(1, H),
                  _rep_spec(1, H), _rep_spec(H, H), _rep_spec(H, H)],
        out_specs=(_row_spec(BN, H), _row_spec(BN, H), _row_spec(BN, H)),
        out_shape=(jax.ShapeDtypeStruct((N, H), _f32),
                   jax.ShapeDtypeStruct((N, H), _f32),
                   jax.ShapeDtypeStruct((N, H), _f32)),
    )(x, p0, p1, wx, wa, b0, w1, b1, g, b, wd, ws)


def _dec(x, p):
    w0, b0, w1, b1 = _mlp_w(p)
    return pl.pallas_call(
        _dec_body,
        grid=(N // BN,),
        in_specs=[_row_spec(BN, H), _rep_spec(H, H), _rep_spec(1, H),
                  _rep_spec(H, DOUT), _rep_spec(1, DOUT)],
        out_specs=_row_spec(BN, DOUT),
        out_shape=jax.ShapeDtypeStruct((N, DOUT), _f32),
    )(x, w0, b0, w1, b1)


def _sc_gather(a, b, dst, src):
    mesh = plsc.VectorSubcoreMesh(core_axis_name="c", subcore_axis_name="s")
    nit = EPW // CH
    npair = nit // 2

    @functools.partial(
        pl.kernel,
        out_type=jax.ShapeDtypeStruct((E, H), _f32),
        mesh=mesh,
        scratch_types=[
            pltpu.VMEM((CH,), jnp.int32),
            pltpu.VMEM((CH,), jnp.int32),
            pltpu.VMEM((CH,), jnp.int32),
            pltpu.VMEM((CH,), jnp.int32),
            pltpu.VMEM((CH, H), _f32),
            pltpu.VMEM((CH, H), _f32),
            pltpu.VMEM((CH, H), _f32),
            pltpu.VMEM((CH, H), _f32),
            pltpu.SemaphoreType.DMA((2,)),
            pltpu.SemaphoreType.DMA((2,)),
            pltpu.SemaphoreType.DMA((2,)),
            pltpu.SemaphoreType.DMA((2,)),
            pltpu.SemaphoreType.DMA((2,)),
        ],
    )
    def k(a_hbm, b_hbm, dst_hbm, src_hbm, g_hbm,
          ia0, ia1, ib0, ib1, ba0, ba1, bb0, bb1,
          sem_ia, sem_ib, sem_ga, sem_gb, sem_o):
        wid = lax.axis_index("s") * NC + lax.axis_index("c")
        base = wid * EPW
        idx_a = (ia0, ia1)
        idx_b = (ib0, ib1)
        buf_a = (ba0, ba1)
        buf_b = (bb0, bb1)

        def idx_cp(g, j):
            off = base + g * CH
            return (pltpu.make_async_copy(dst_hbm.at[pl.ds(off, CH)],
                                          idx_a[j], sem_ia.at[j]),
                    pltpu.make_async_copy(src_hbm.at[pl.ds(off, CH)],
                                          idx_b[j], sem_ib.at[j]))

        def gat_cp(j):
            return (pltpu.make_async_copy(a_hbm.at[idx_a[j]],
                                          buf_a[j], sem_ga.at[j]),
                    pltpu.make_async_copy(b_hbm.at[idx_b[j]],
                                          buf_b[j], sem_gb.at[j]))

        def out_cp(g, j):
            off = base + g * CH
            return (pltpu.make_async_copy(buf_a[j],
                                          g_hbm.at[pl.ds(off, CH)],
                                          sem_o.at[j]),)

        def start(cps):
            for cp in cps:
                cp.start()

        def wait(cps):
            for cp in cps:
                cp.wait()

        def add_inplace(j):
            av = buf_a[j]
            bv = buf_b[j]

            def row(r, carry):
                for cseg in range(H // 16):
                    sl = pl.ds(cseg * 16, 16)
                    av[r, sl] = av[r, sl] + bv[r, sl]
                return carry

            lax.fori_loop(0, CH, row, 0)

        start(idx_cp(0, 0))
        start(idx_cp(1, 1))
        wait(idx_cp(0, 0))
        start(gat_cp(0))
        wait(idx_cp(1, 1))
        start(gat_cp(1))

        def body(i, carry):
            g0 = 2 * i
            g1 = g0 + 1

            wait(gat_cp(0))
            @pl.when(g0 + 2 < nit)
            def _():
                start(idx_cp(g0 + 2, 0))
            add_inplace(0)
            start(out_cp(g0, 0))

            wait(gat_cp(1))
            @pl.when(g1 + 2 < nit)
            def _():
                start(idx_cp(g1 + 2, 1))
            add_inplace(1)
            start(out_cp(g1, 1))

            wait(out_cp(g0, 0))
            @pl.when(g0 + 2 < nit)
            def _():
                wait(idx_cp(g0 + 2, 0))
                start(gat_cp(0))
            wait(out_cp(g1, 1))
            @pl.when(g1 + 2 < nit)
            def _():
                wait(idx_cp(g1 + 2, 1))
                start(gat_cp(1))
            return carry

        lax.fori_loop(0, npair, body, 0)

    return k(a, b, dst, src)


def _sc_scatter(ue, src, zrows):
    mesh = plsc.VectorSubcoreMesh(core_axis_name="c", subcore_axis_name="s")
    nit = EPW // CHS
    npair = nit // 2

    @functools.partial(
        pl.kernel,
        out_type=jax.ShapeDtypeStruct((NC, NPAD, H), _f32),
        mesh=mesh,
        scratch_types=[
            pltpu.VMEM((CHS,), jnp.int32),
            pltpu.VMEM((CHS,), jnp.int32),
            pltpu.VMEM((CHS, H), _f32),
            pltpu.VMEM((CHS, H), _f32),
            pltpu.VMEM_SHARED((NPAD, H), _f32),
            pltpu.SemaphoreType.DMA((2,)),
            pltpu.SemaphoreType.DMA((2,)),
        ],
    )
    def k(ue_hbm, src_hbm, z_hbm, out_hbm, ix0, ix1, bf0, bf1, acc_sh,
          sem_i, sem_u):
        c = lax.axis_index("c")
        s = lax.axis_index("s")
        r0 = s * RPT
        pltpu.sync_copy(z_hbm, acc_sh.at[pl.ds(r0, RPT)])
        plsc.subcore_barrier()
        base = c * (E // NC) + s * EPW
        idx_v = (ix0, ix1)
        buf = (bf0, bf1)

        def load_cp(g, j):
            return (pltpu.make_async_copy(src_hbm.at[pl.ds(base + g * CHS, CHS)],
                                          idx_v[j], sem_i.at[j]),
                    pltpu.make_async_copy(ue_hbm.at[pl.ds(base + g * CHS, CHS)],
                                          buf[j], sem_u.at[j]))

        def start(cps):
            for cp in cps:
                cp.start()

        def wait(cps):
            for cp in cps:
                cp.wait()

        start(load_cp(0, 0))

        def body(i, carry):
            g0 = 2 * i
            g1 = g0 + 1
            start(load_cp(g1, 1))
            wait(load_cp(g0, 0))
            pltpu.sync_copy(buf[0], acc_sh.at[idx_v[0]], add=True)

            @pl.when(g1 + 1 < nit)
            def _():
                start(load_cp(g1 + 1, 0))
            wait(load_cp(g1, 1))
            pltpu.sync_copy(buf[1], acc_sh.at[idx_v[1]], add=True)
            return carry

        lax.fori_loop(0, npair, body, 0)
        if nit % 2 == 1:
            wait(load_cp(nit - 1, 0))
            pltpu.sync_copy(buf[0], acc_sh.at[idx_v[0]], add=True)
        plsc.subcore_barrier()
        pltpu.sync_copy(acc_sh.at[pl.ds(r0, RPT)],
                        out_hbm.at[c, pl.ds(r0, RPT)])

    return k(ue, src, zrows)


def kernel(x, edge_index, edge_attr, mean_vec_x, std_vec_x,
           mean_vec_edge, std_vec_edge, params):
    src = edge_index[0].astype(jnp.int32)
    dst = edge_index[1].astype(jnp.int32)

    lw = []
    for lp in params["layers"]:
        w0e = lp["edge_mlp"]["l0"]["w"]
        lw.append((w0e[:H], w0e[H:2 * H], w0e[2 * H:]))

    xc, a, b = _node_enc(x, mean_vec_x, std_vec_x, params["node_enc"],
                         lw[0][0], lw[0][1])
    ea = None
    zrows = jnp.zeros((RPT, H), _f32)

    nl = len(params["layers"])
    for li, lp in enumerate(params["layers"]):
        ew = lp["edge_mlp"]
        we = lw[li][2]
        b0e = ew["l0"]["b"].reshape(1, H)
        w1e, b1e = ew["l1"]["w"], ew["l1"]["b"].reshape(1, H)
        ge, be = ew["ln"]["g"].reshape(1, H), ew["ln"]["b"].reshape(1, H)

        gsum = _sc_gather(a, b, dst, src)
        if li == 0:
            ue = _edge_upd0(edge_attr, mean_vec_edge, std_vec_edge,
                            params["edge_enc"], we, b0e, w1e, b1e,
                            ge, be)(gsum)
        else:
            ue = _edge_upd(gsum, ea, we, b0e, w1e, b1e, ge, be)
        parts = _sc_scatter(ue, src, zrows)
        p0, p1 = parts[0, :N], parts[1, :N]

        nw = lp["node_mlp"]
        w0n = nw["l0"]["w"]
        wx, wa = w0n[:H], w0n[H:]
        b0n = nw["l0"]["b"].reshape(1, H)
        w1n, b1n = nw["l1"]["w"], nw["l1"]["b"].reshape(1, H)
        gn, bn = nw["ln"]["g"].reshape(1, H), nw["ln"]["b"].reshape(1, H)
        nwd, nws = lw[(li + 1) % nl][0], lw[(li + 1) % nl][1]
        xc, a, b = _node_upd(xc, p0, p1, wx, wa, b0n, w1n, b1n, gn, bn,
                             nwd, nws)
        ea = ue

    return _dec(xc, params["dec"])
